# Initial kernel scaffold; baseline (speedup 1.0000x reference)
import jax
import jax.numpy as jnp
from jax.experimental import pallas as pl


def kernel(x, edge_index, Wl1, bl1, Wr1, Wl2, bl2, Wr2, Vl1, vb1, Vr1, Vl2, vb2, Vr2, Wc1, bc1, Wc2, bc2):
    raise NotImplementedError("write your pallas kernel here")


# speedup vs baseline: 3.7289x; 3.7289x over previous
import functools

import jax
import jax.numpy as jnp
from jax import lax
from jax.experimental import pallas as pl
from jax.experimental.pallas import tpu as pltpu
from jax.experimental.pallas import tpu_sc as plsc

NC = 2
NS = 16
NW = NC * NS
CH = 128
RB = 400


def _sc_agg(table, src1, dst1, zeros_d, npad):
    ne = src1.shape[0]
    k = ne // (NW * CH)
    d = table.shape[1]
    rp = npad // NS
    mesh = plsc.VectorSubcoreMesh(core_axis_name="c", subcore_axis_name="s",
                                  num_cores=NC, num_subcores=NS)
    scratch = [
        pltpu.VMEM((CH,), jnp.int32),
        pltpu.VMEM((CH,), jnp.int32),
        pltpu.VMEM((CH, d), jnp.float32),
        pltpu.VMEM_SHARED((npad, d), jnp.float32),
        pltpu.SemaphoreType.DMA,
    ]

    @functools.partial(
        pl.kernel, out_type=jax.ShapeDtypeStruct((NC * npad, d), jnp.float32),
        mesh=mesh, scratch_types=scratch)
    def k_fn(table_h, src_h, dst_h, zd_h, sum_h, sidx, didx, rows, acc, sem):
        c = lax.axis_index("c")
        s = lax.axis_index("s")
        wid = s * NC + c

        pltpu.sync_copy(zd_h, acc.at[pl.ds(s * rp, rp)])
        plsc.subcore_barrier()

        def chunk(j, cc):
            base = (wid * k + j) * CH
            pltpu.sync_copy(src_h.at[pl.ds(base, CH)], sidx)
            pltpu.sync_copy(dst_h.at[pl.ds(base, CH)], didx)
            pltpu.async_copy(table_h.at[sidx], rows, sem).wait()
            pltpu.sync_copy(rows, acc.at[didx], add=True)
            return cc
        lax.fori_loop(0, k, chunk, 0)

        plsc.subcore_barrier()
        pltpu.sync_copy(acc.at[pl.ds(s * rp, rp)],
                        sum_h.at[pl.ds(c * npad + s * rp, rp)])

    return k_fn(table, src1, dst1, zeros_d)


def _sc_count(dst1, zeros_d, ones_d, npad):
    ne = dst1.shape[0]
    k = ne // (NW * CH)
    d = ones_d.shape[1]
    rp = npad // NS
    mesh = plsc.VectorSubcoreMesh(core_axis_name="c", subcore_axis_name="s",
                                  num_cores=NC, num_subcores=NS)
    scratch = [
        pltpu.VMEM((CH,), jnp.int32),
        pltpu.VMEM((CH, d), jnp.float32),
        pltpu.VMEM_SHARED((npad, d), jnp.float32),
    ]

    @functools.partial(
        pl.kernel, out_type=jax.ShapeDtypeStruct((NC * npad, d), jnp.float32),
        mesh=mesh, scratch_types=scratch)
    def k_fn(dst_h, zd_h, od_h, cnt_h, didx, ones_v, acc):
        c = lax.axis_index("c")
        s = lax.axis_index("s")
        wid = s * NC + c
        pltpu.sync_copy(zd_h, acc.at[pl.ds(s * rp, rp)])
        pltpu.sync_copy(od_h, ones_v)
        plsc.subcore_barrier()

        def chunk(j, cc):
            base = (wid * k + j) * CH
            pltpu.sync_copy(dst_h.at[pl.ds(base, CH)], didx)
            pltpu.sync_copy(ones_v, acc.at[didx], add=True)
            return cc
        lax.fori_loop(0, k, chunk, 0)

        plsc.subcore_barrier()
        pltpu.sync_copy(acc.at[pl.ds(s * rp, rp)],
                        cnt_h.at[pl.ds(c * npad + s * rp, rp)])

    return k_fn(dst1, zeros_d, ones_d)


def _dot(a, b):
    return jnp.dot(a, b, preferred_element_type=jnp.float32)


def _tc1_body(sum_ref, cnt_ref, x_ref, wl, bl, wr, vl, vb, vr, l1_ref, g1_ref):
    scount = jnp.maximum(cnt_ref[0, :, 0:1] + cnt_ref[1, :, 0:1], 1.0)
    mean = (sum_ref[0] + sum_ref[1]) / scount
    xb = x_ref[...]
    l1_ref[...] = jnp.maximum(
        _dot(mean, wl[...]) + bl[...] + _dot(xb, wr[...]), 0.0)
    g1_ref[...] = jnp.maximum(
        _dot(mean, vl[...]) + vb[...] + _dot(xb, vr[...]), 0.0)


def _tc2_body(suml_ref, sumg_ref, cnt_ref, l1_ref, g1_ref,
              wl, bl, wr, vl, vb, vr, wc1a, wc1b, bc1, wc2, bc2, out_ref):
    scount = jnp.maximum(cnt_ref[0, :, 0:1] + cnt_ref[1, :, 0:1], 1.0)
    ml = (suml_ref[0] + suml_ref[1]) / scount
    mg = (sumg_ref[0] + sumg_ref[1]) / scount
    l2 = _dot(ml, wl[...]) + bl[...] + _dot(l1_ref[...], wr[...])
    g2 = _dot(mg, vl[...]) + vb[...] + _dot(g1_ref[...], vr[...])
    h = jnp.maximum(_dot(l2, wc1a[...]) + _dot(g2, wc1b[...]) + bc1[...], 0.0)
    out_ref[...] = _dot(h, wc2[...]) + bc2[...]


def kernel(x, edge_index, Wl1, bl1, Wr1, Wl2, bl2, Wr2,
           Vl1, vb1, Vr1, Vl2, vb2, Vr2, Wc1, bc1, Wc2, bc2):
    n, d = x.shape
    e = edge_index.shape[1]
    o = Wc2.shape[0]
    h = Wl1.shape[0]

    epw = CH * NW
    epad = -(-e // epw) * epw
    npad = -(-n // (NS * 8)) * (NS * 8)
    if npad == n:
        npad += NS * 8
    rp = npad // NS

    src = edge_index[0]
    dst = edge_index[1]
    pad = epad - e
    src1 = jnp.concatenate([src, jnp.zeros((pad,), jnp.int32)])
    dst1 = jnp.concatenate(
        [dst, n + (jnp.arange(pad, dtype=jnp.int32) % (npad - n))])

    zeros_d = jnp.zeros((rp, d), jnp.float32)
    ones_d = jnp.zeros((CH, d), jnp.float32).at[:, 0].set(1.0)

    sum_x = _sc_agg(x, src1, dst1, zeros_d, npad).reshape(NC, npad, d)
    cnt = _sc_count(dst1, zeros_d, ones_d, npad).reshape(NC, npad, d)

    grid = (n // RB,)
    wspec = pl.BlockSpec((d, h), lambda i: (0, 0))
    bspec = pl.BlockSpec((1, h), lambda i: (0, 0))
    rowspec = pl.BlockSpec((RB, d), lambda i: (i, 0))
    sumspec = pl.BlockSpec((NC, RB, d), lambda i: (0, i, 0))
    l1, g1 = pl.pallas_call(
        _tc1_body,
        grid=grid,
        in_specs=[sumspec, sumspec, rowspec,
                  wspec, bspec, wspec, wspec, bspec, wspec],
        out_specs=[rowspec, rowspec],
        out_shape=[jax.ShapeDtypeStruct((n, h), jnp.float32)] * 2,
    )(sum_x, cnt, x,
      Wl1.T, bl1.reshape(1, h), Wr1.T, Vl1.T, vb1.reshape(1, h), Vr1.T)

    sum_l = _sc_agg(l1, src1, dst1, zeros_d, npad).reshape(NC, npad, d)
    sum_g = _sc_agg(g1, src1, dst1, zeros_d, npad).reshape(NC, npad, d)

    wc2p = jnp.zeros((h, 128), jnp.float32).at[:, :o].set(Wc2.T)
    bc2p = jnp.zeros((1, 128), jnp.float32).at[0, :o].set(bc2)
    outspec = pl.BlockSpec((RB, 128), lambda i: (i, 0))
    outp = pl.pallas_call(
        _tc2_body,
        grid=grid,
        in_specs=[sumspec, sumspec, sumspec, rowspec, rowspec,
                  wspec, bspec, wspec, wspec, bspec, wspec,
                  wspec, wspec, bspec,
                  pl.BlockSpec((h, 128), lambda i: (0, 0)), bspec],
        out_specs=outspec,
        out_shape=jax.ShapeDtypeStruct((n, 128), jnp.float32),
    )(sum_l, sum_g, cnt, l1, g1,
      Wl2.T, bl2.reshape(1, h), Wr2.T, Vl2.T, vb2.reshape(1, h), Vr2.T,
      Wc1[:, :h].T, Wc1[:, h:].T, bc1.reshape(1, h), wc2p, bc2p)
    return outp[:, :o]


# speedup vs baseline: 4.4798x; 1.2014x over previous
import functools

import jax
import jax.numpy as jnp
from jax import lax
from jax.experimental import pallas as pl
from jax.experimental.pallas import tpu as pltpu
from jax.experimental.pallas import tpu_sc as plsc

NC = 2
NS = 16
NW = NC * NS
CH = 128
RB = 400


def _sc_agg(table, src1, dst1, zeros_d, npad):
    ne = src1.shape[0]
    k = ne // (NW * CH)
    d = table.shape[1]
    rp = npad // NS
    mesh = plsc.VectorSubcoreMesh(core_axis_name="c", subcore_axis_name="s",
                                  num_cores=NC, num_subcores=NS)
    scratch = [
        pltpu.VMEM((CH,), jnp.int32),
        pltpu.VMEM((CH,), jnp.int32),
        pltpu.VMEM((CH,), jnp.int32),
        pltpu.VMEM((CH,), jnp.int32),
        pltpu.VMEM((CH, d), jnp.float32),
        pltpu.VMEM((CH, d), jnp.float32),
        pltpu.VMEM_SHARED((npad, d), jnp.float32),
        pltpu.SemaphoreType.DMA,
        pltpu.SemaphoreType.DMA,
    ]

    @functools.partial(
        pl.kernel, out_type=jax.ShapeDtypeStruct((NC * npad, d), jnp.float32),
        mesh=mesh, scratch_types=scratch)
    def k_fn(table_h, src_h, dst_h, zd_h, sum_h,
             sidx_a, didx_a, sidx_b, didx_b, rows_a, rows_b, acc,
             sem_a, sem_b):
        c = lax.axis_index("c")
        s = lax.axis_index("s")
        wid = s * NC + c

        def stage(j, sidx, didx):
            base = (wid * k + jnp.minimum(j, k - 1)) * CH
            pltpu.sync_copy(src_h.at[pl.ds(base, CH)], sidx)
            pltpu.sync_copy(dst_h.at[pl.ds(base, CH)], didx)

        pltpu.sync_copy(zd_h, acc.at[pl.ds(s * rp, rp)])
        plsc.subcore_barrier()

        stage(0, sidx_a, didx_a)
        pltpu.async_copy(table_h.at[sidx_a], rows_a, sem_a)

        def pair(i, cc):
            j0 = 2 * i
            stage(j0 + 1, sidx_b, didx_b)
            pltpu.make_async_copy(table_h.at[sidx_a], rows_a, sem_a).wait()
            pltpu.async_copy(table_h.at[sidx_b], rows_b, sem_b)
            pltpu.sync_copy(rows_a, acc.at[didx_a], add=True)
            stage(j0 + 2, sidx_a, didx_a)
            pltpu.make_async_copy(table_h.at[sidx_b], rows_b, sem_b).wait()
            pltpu.async_copy(table_h.at[sidx_a], rows_a, sem_a)

            @pl.when(j0 + 1 <= k - 1)
            def _():
                pltpu.sync_copy(rows_b, acc.at[didx_b], add=True)
            return cc
        lax.fori_loop(0, (k + 1) // 2, pair, 0)
        pltpu.make_async_copy(table_h.at[sidx_a], rows_a, sem_a).wait()

        plsc.subcore_barrier()
        pltpu.sync_copy(acc.at[pl.ds(s * rp, rp)],
                        sum_h.at[pl.ds(c * npad + s * rp, rp)])

    return k_fn(table, src1, dst1, zeros_d)


def _sc_count(dst1, zeros_d, ones_d, npad):
    ne = dst1.shape[0]
    k = ne // (NW * CH)
    d = ones_d.shape[1]
    rp = npad // NS
    mesh = plsc.VectorSubcoreMesh(core_axis_name="c", subcore_axis_name="s",
                                  num_cores=NC, num_subcores=NS)
    scratch = [
        pltpu.VMEM((CH,), jnp.int32),
        pltpu.VMEM((CH, d), jnp.float32),
        pltpu.VMEM_SHARED((npad, d), jnp.float32),
    ]

    @functools.partial(
        pl.kernel, out_type=jax.ShapeDtypeStruct((NC * npad, d), jnp.float32),
        mesh=mesh, scratch_types=scratch)
    def k_fn(dst_h, zd_h, od_h, cnt_h, didx, ones_v, acc):
        c = lax.axis_index("c")
        s = lax.axis_index("s")
        wid = s * NC + c
        pltpu.sync_copy(zd_h, acc.at[pl.ds(s * rp, rp)])
        pltpu.sync_copy(od_h, ones_v)
        plsc.subcore_barrier()

        def chunk(j, cc):
            base = (wid * k + j) * CH
            pltpu.sync_copy(dst_h.at[pl.ds(base, CH)], didx)
            pltpu.sync_copy(ones_v, acc.at[didx], add=True)
            return cc
        lax.fori_loop(0, k, chunk, 0)

        plsc.subcore_barrier()
        pltpu.sync_copy(acc.at[pl.ds(s * rp, rp)],
                        cnt_h.at[pl.ds(c * npad + s * rp, rp)])

    return k_fn(dst1, zeros_d, ones_d)


def _dot(a, b):
    return jnp.dot(a, b, preferred_element_type=jnp.float32)


def _tc1_body(sum_ref, cnt_ref, x_ref, wl, bl, wr, vl, vb, vr, l1_ref, g1_ref):
    scount = jnp.maximum(cnt_ref[0, :, 0:1] + cnt_ref[1, :, 0:1], 1.0)
    mean = (sum_ref[0] + sum_ref[1]) / scount
    xb = x_ref[...]
    l1_ref[...] = jnp.maximum(
        _dot(mean, wl[...]) + bl[...] + _dot(xb, wr[...]), 0.0)
    g1_ref[...] = jnp.maximum(
        _dot(mean, vl[...]) + vb[...] + _dot(xb, vr[...]), 0.0)


def _tc2_body(suml_ref, sumg_ref, cnt_ref, l1_ref, g1_ref,
              wl, bl, wr, vl, vb, vr, wc1a, wc1b, bc1, wc2, bc2, out_ref):
    scount = jnp.maximum(cnt_ref[0, :, 0:1] + cnt_ref[1, :, 0:1], 1.0)
    ml = (suml_ref[0] + suml_ref[1]) / scount
    mg = (sumg_ref[0] + sumg_ref[1]) / scount
    l2 = _dot(ml, wl[...]) + bl[...] + _dot(l1_ref[...], wr[...])
    g2 = _dot(mg, vl[...]) + vb[...] + _dot(g1_ref[...], vr[...])
    h = jnp.maximum(_dot(l2, wc1a[...]) + _dot(g2, wc1b[...]) + bc1[...], 0.0)
    out_ref[...] = _dot(h, wc2[...]) + bc2[...]


def kernel(x, edge_index, Wl1, bl1, Wr1, Wl2, bl2, Wr2,
           Vl1, vb1, Vr1, Vl2, vb2, Vr2, Wc1, bc1, Wc2, bc2):
    n, d = x.shape
    e = edge_index.shape[1]
    o = Wc2.shape[0]
    h = Wl1.shape[0]

    epw = CH * NW
    epad = -(-e // epw) * epw
    npad = -(-n // (NS * 8)) * (NS * 8)
    if npad == n:
        npad += NS * 8
    rp = npad // NS

    src = edge_index[0]
    dst = edge_index[1]
    pad = epad - e
    src1 = jnp.concatenate([src, jnp.zeros((pad,), jnp.int32)])
    dst1 = jnp.concatenate(
        [dst, n + (jnp.arange(pad, dtype=jnp.int32) % (npad - n))])

    zeros_d = jnp.zeros((rp, d), jnp.float32)
    ones_d = jnp.zeros((CH, d), jnp.float32).at[:, 0].set(1.0)

    sum_x = _sc_agg(x, src1, dst1, zeros_d, npad).reshape(NC, npad, d)
    cnt = _sc_count(dst1, zeros_d, ones_d, npad).reshape(NC, npad, d)

    grid = (n // RB,)
    wspec = pl.BlockSpec((d, h), lambda i: (0, 0))
    bspec = pl.BlockSpec((1, h), lambda i: (0, 0))
    rowspec = pl.BlockSpec((RB, d), lambda i: (i, 0))
    sumspec = pl.BlockSpec((NC, RB, d), lambda i: (0, i, 0))
    l1, g1 = pl.pallas_call(
        _tc1_body,
        grid=grid,
        in_specs=[sumspec, sumspec, rowspec,
                  wspec, bspec, wspec, wspec, bspec, wspec],
        out_specs=[rowspec, rowspec],
        out_shape=[jax.ShapeDtypeStruct((n, h), jnp.float32)] * 2,
    )(sum_x, cnt, x,
      Wl1.T, bl1.reshape(1, h), Wr1.T, Vl1.T, vb1.reshape(1, h), Vr1.T)

    sum_l = _sc_agg(l1, src1, dst1, zeros_d, npad).reshape(NC, npad, d)
    sum_g = _sc_agg(g1, src1, dst1, zeros_d, npad).reshape(NC, npad, d)

    wc2p = jnp.zeros((h, 128), jnp.float32).at[:, :o].set(Wc2.T)
    bc2p = jnp.zeros((1, 128), jnp.float32).at[0, :o].set(bc2)
    outspec = pl.BlockSpec((RB, 128), lambda i: (i, 0))
    outp = pl.pallas_call(
        _tc2_body,
        grid=grid,
        in_specs=[sumspec, sumspec, sumspec, rowspec, rowspec,
                  wspec, bspec, wspec, wspec, bspec, wspec,
                  wspec, wspec, bspec,
                  pl.BlockSpec((h, 128), lambda i: (0, 0)), bspec],
        out_specs=outspec,
        out_shape=jax.ShapeDtypeStruct((n, 128), jnp.float32),
    )(sum_l, sum_g, cnt, l1, g1,
      Wl2.T, bl2.reshape(1, h), Wr2.T, Vl2.T, vb2.reshape(1, h), Vr2.T,
      Wc1[:, :h].T, Wc1[:, h:].T, bc1.reshape(1, h), wc2p, bc2p)
    return outp[:, :o]
